# baseline (speedup 1.0000x reference)
import functools

import jax
import jax.numpy as jnp
from jax import lax
from jax.experimental import pallas as pl
from jax.experimental.pallas import tpu as pltpu
from jax.experimental.pallas import tpu_sc as plsc

B = 128
V = 100000
K64 = 64
NEG = -1e30
EPS = 1e-8
NINF = float("-inf")
PINF = float("inf")


WA = 4096
NA = -(-V // WA)


def _stats_body(x_ref, m_ref, z_ref, ms, zs):
    pid = pl.program_id(0)

    @pl.when(pid == 0)
    def _():
        ms[...] = jnp.full_like(ms[...], NINF)
        zs[...] = jnp.zeros_like(zs[...])

    x = x_ref[...]
    cols = pid * WA + lax.broadcasted_iota(jnp.int32, (B, WA), 1)
    xv = jnp.where(cols < V, x, NINF)
    bm = jnp.max(xv, axis=1, keepdims=True)
    m_old = ms[...]
    m_new = jnp.maximum(m_old, bm)
    bz = jnp.sum(jnp.exp(xv - m_new), axis=1, keepdims=True)
    zs[...] = zs[...] * jnp.exp(m_old - m_new) + bz
    ms[...] = m_new
    m_ref[...] = ms[...]
    z_ref[...] = zs[...]


def _row_stats(logits):
    return pl.pallas_call(
        _stats_body,
        grid=(NA,),
        in_specs=[pl.BlockSpec((B, WA), lambda i: (0, i))],
        out_specs=[
            pl.BlockSpec((B, 1), lambda i: (0, 0)),
            pl.BlockSpec((B, 1), lambda i: (0, 0)),
        ],
        out_shape=[
            jax.ShapeDtypeStruct((B, 1), jnp.float32),
            jax.ShapeDtypeStruct((B, 1), jnp.float32),
        ],
        scratch_shapes=[
            pltpu.VMEM((B, 1), jnp.float32),
            pltpu.VMEM((B, 1), jnp.float32),
        ],
    )(logits)


NC = 2
NS = 16
NW = NC * NS
ROWS_PER = B // NW
GRP = 10
NGRP = V // (16 * GRP)
APG = 3
APV = APG * GRP
AP0 = K64
CANDV = (AP0 // 16) + APV
CAND = CANDV * 16


def _scalar(vec):
    return lax.squeeze(lax.slice(vec, (0,), (1,)), dimensions=(0,))


def _splat_f(s):
    return jnp.full((16,), s, jnp.float32)


def _splat_i(s):
    return jnp.full((16,), s, jnp.int32)


def _topk_body(logits_hbm, tops_hbm, topi_hbm, buf, cand, candi,
               topsv, topsi):
    wid = lax.axis_index("s") * NC + lax.axis_index("c")
    lane = lax.iota(jnp.int32, 16)

    def _consolidate(_):
        def find_body(j, carry):
            t_cur, rank, t64 = carry
            tc = _splat_f(t_cur)

            def mx_body(k, m):
                v = cand[pl.ds(k * 16, 16)]
                return jnp.maximum(m, jnp.where(v < tc, v, _splat_f(NINF)))

            nm = jnp.max(lax.fori_loop(0, CANDV, mx_body, _splat_f(NINF)))
            nms = _splat_f(nm)

            def cnt_body(k, s):
                v = cand[pl.ds(k * 16, 16)]
                return s + jnp.where(v == nms, 1, 0).astype(jnp.int32)

            c = jnp.sum(lax.fori_loop(0, CANDV, cnt_body, _splat_i(0)))
            t64 = jnp.where(rank < K64, nm, t64)
            return nm, rank + c, t64

        _, _, t64 = lax.fori_loop(
            0, K64, find_body,
            (jnp.float32(PINF), jnp.int32(0), jnp.float32(NINF)))
        t64s = _splat_f(t64)

        def g_body(k, ge):
            g, e = ge
            v = cand[pl.ds(k * 16, 16)]
            g = g + jnp.where(v > t64s, 1, 0).astype(jnp.int32)
            e = e + jnp.where(v == t64s, 1, 0).astype(jnp.int32)
            return g, e

        g, e = lax.fori_loop(0, CANDV, g_body, (_splat_i(0), _splat_i(0)))
        skip = jnp.sum(e) - (K64 - jnp.sum(g))

        def comp_body(k, carry):
            w, eq_seen = carry
            v = cand[pl.ds(k * 16, 16)]
            iv = candi[pl.ds(k * 16, 16)]
            gm = v > t64s
            em = v == t64s
            pos = plsc.cumsum(jnp.where(em, 1, 0).astype(jnp.int32))
            pos = pos + _splat_i(eq_seen)
            take = jnp.logical_or(
                gm, jnp.logical_and(em, pos > _splat_i(skip)))
            plsc.store_compressed(cand.at[pl.ds(w, 16)], v, mask=take)
            plsc.store_compressed(candi.at[pl.ds(w, 16)], iv, mask=take)
            nw = w + jnp.sum(jnp.where(take, 1, 0).astype(jnp.int32))
            return nw, eq_seen + jnp.sum(jnp.where(em, 1, 0).astype(jnp.int32))

        lax.fori_loop(0, CANDV, comp_body, (jnp.int32(0), jnp.int32(0)))

        def clr_body(k, c):
            cand[pl.ds(AP0 + k * 16, 16)] = _splat_f(NINF)
            return c

        lax.fori_loop(0, APV, clr_body, jnp.int32(0))
        return t64

    def row_body(r, c0):
        row = wid * ROWS_PER + r

        def initc(k, c):
            cand[pl.ds(k * 16, 16)] = _splat_f(NINF)
            return c

        lax.fori_loop(0, CANDV, initc, jnp.int32(0))
        pltpu.sync_copy(logits_hbm.at[pl.ds(row * V, V)], buf)

        def scan_body(i, carry):
            tv, cnt = carry
            vs = [buf[pl.ds((i * GRP + u) * 16, 16)] for u in range(GRP)]
            m = vs[0]
            for u in range(1, GRP):
                m = jnp.maximum(m, vs[u])
            hit = jnp.any(m >= tv)

            def do_append(tv, cnt):
                for u in range(GRP):
                    cand[pl.ds(AP0 + (cnt + u) * 16, 16)] = vs[u]
                    candi[pl.ds(AP0 + (cnt + u) * 16, 16)] = (
                        _splat_i((i * GRP + u) * 16) + lane)
                cnt = cnt + GRP

                def run_cons(_):
                    return _splat_f(_consolidate(0)), jnp.int32(0)

                return lax.cond(cnt >= APV, run_cons,
                                lambda _: (tv, cnt), 0)

            return lax.cond(hit, do_append, lambda tv, cnt: (tv, cnt),
                            tv, cnt)

        lax.fori_loop(0, NGRP, scan_body, (_splat_f(NINF), jnp.int32(0)))
        _consolidate(0)

        def ex_body(j, carry):
            cur, rem, pidx = carry

            def adv(_):
                cs = _splat_f(cur)
                m = _splat_f(NINF)
                for k in range(K64 // 16):
                    v = cand[pl.ds(k * 16, 16)]
                    m = jnp.maximum(m, jnp.where(v < cs, v, _splat_f(NINF)))
                nm = jnp.max(m)
                nms = _splat_f(nm)
                cvec = _splat_i(0)
                for k in range(K64 // 16):
                    v = cand[pl.ds(k * 16, 16)]
                    cvec = cvec + jnp.where(v == nms, 1, 0).astype(jnp.int32)
                return nm, jnp.sum(cvec), jnp.int32(2 ** 30)

            cur, rem, pidx = lax.cond(rem == 0, adv,
                                      lambda _: (cur, rem, pidx), 0)
            curs = _splat_f(cur)
            pv = _splat_i(pidx)
            mi = _splat_i(-1)
            for k in range(K64 // 16):
                v = cand[pl.ds(k * 16, 16)]
                iv = candi[pl.ds(k * 16, 16)]
                sel = jnp.logical_and(v == curs, iv < pv)
                mi = jnp.maximum(mi, jnp.where(sel, iv, _splat_i(-1)))
            eidx = jnp.max(mi)
            plsc.store_scatter(topsv, [_splat_i(j)], curs, mask=lane == 0)
            plsc.store_scatter(topsi, [_splat_i(j)], _splat_i(eidx),
                               mask=lane == 0)
            return cur, rem - 1, eidx

        lax.fori_loop(0, K64, ex_body,
                      (jnp.float32(PINF), jnp.int32(0), jnp.int32(2 ** 30)))
        pltpu.sync_copy(topsv, tops_hbm.at[pl.ds(row * K64, K64)])
        pltpu.sync_copy(topsi, topi_hbm.at[pl.ds(row * K64, K64)])
        return c0

    lax.fori_loop(0, ROWS_PER, row_body, jnp.int32(0))


def _row_tops(logits_flat):
    mesh = plsc.VectorSubcoreMesh(
        core_axis_name="c", subcore_axis_name="s",
        num_cores=NC, num_subcores=NS)
    return pl.kernel(
        _topk_body,
        out_type=(jax.ShapeDtypeStruct((B * K64,), jnp.float32),
                  jax.ShapeDtypeStruct((B * K64,), jnp.int32)),
        mesh=mesh,
        compiler_params=pltpu.CompilerParams(needs_layout_passes=False),
        scratch_types=[
            pltpu.VMEM((V,), jnp.float32),
            pltpu.VMEM((CAND,), jnp.float32),
            pltpu.VMEM((CAND,), jnp.int32),
            pltpu.VMEM((K64,), jnp.float32),
            pltpu.VMEM((K64,), jnp.int32),
        ],
    )(logits_flat)


WD = 4096
ND = -(-V // WD)


def _final_body(tops_ref, topi_ref, m_ref, z_ref, ks_ref, ps_ref, x_ref,
                q_ref, out_ref, idx_ref, ts, cs_s, bv, bi):
    pid = pl.program_id(0)

    @pl.when(pid == 0)
    def _():
        tops = tops_ref[...]
        topi = topi_ref[...]
        M = m_ref[...]
        Z = z_ref[...]
        p = jnp.exp(tops - M) / Z
        r = lax.broadcasted_iota(jnp.int32, (K64, K64), 0)
        c = lax.broadcasted_iota(jnp.int32, (K64, K64), 1)
        ut = (r < c).astype(jnp.float32)
        excl = jnp.dot(p, ut, preferred_element_type=jnp.float32)
        ranks = lax.broadcasted_iota(jnp.int32, (B, K64), 1)
        keep = (ranks < ks_ref[...]) & (excl < ps_ref[...])
        m = jnp.sum(keep.astype(jnp.int32), axis=1, keepdims=True)
        last = ranks == (m - 1)
        ts[...] = jnp.max(jnp.where(last, tops, NINF), axis=1, keepdims=True)
        cs_s[...] = jnp.max(jnp.where(last, topi, -1), axis=1, keepdims=True)
        bv[...] = jnp.full((B, 1), -1.0, jnp.float32)
        bi[...] = jnp.zeros((B, 1), jnp.int32)

    x = x_ref[...]
    q = q_ref[...]
    cols = pid * WD + lax.broadcasted_iota(jnp.int32, (B, WD), 1)
    tsv = ts[...]
    kept = ((x > tsv) | ((x == tsv) & (cols >= cs_s[...]))) & (cols < V)
    out_ref[...] = jnp.where(kept, x, NEG)
    val = jnp.where(kept, jnp.exp(x - m_ref[...]) / jnp.maximum(q, EPS),
                    jnp.float32(-1.0))
    cmax = jnp.max(val, axis=1, keepdims=True)
    cidx = jnp.min(jnp.where(val == cmax, cols, jnp.int32(V)),
                   axis=1, keepdims=True)
    better = cmax > bv[...]
    bv[...] = jnp.where(better, cmax, bv[...])
    bi[...] = jnp.where(better, cidx, bi[...])
    idx_ref[...] = bi[...]


def _finalize(tops, topi, M, Z, ks, ps, logits, q):
    return pl.pallas_call(
        _final_body,
        grid=(ND,),
        in_specs=[
            pl.BlockSpec((B, K64), lambda i: (0, 0)),
            pl.BlockSpec((B, K64), lambda i: (0, 0)),
            pl.BlockSpec((B, 1), lambda i: (0, 0)),
            pl.BlockSpec((B, 1), lambda i: (0, 0)),
            pl.BlockSpec((B, 1), lambda i: (0, 0)),
            pl.BlockSpec((B, 1), lambda i: (0, 0)),
            pl.BlockSpec((B, WD), lambda i: (0, i)),
            pl.BlockSpec((B, WD), lambda i: (0, i)),
        ],
        out_specs=[
            pl.BlockSpec((B, WD), lambda i: (0, i)),
            pl.BlockSpec((B, 1), lambda i: (0, 0)),
        ],
        out_shape=[
            jax.ShapeDtypeStruct((B, V), jnp.float32),
            jax.ShapeDtypeStruct((B, 1), jnp.int32),
        ],
        scratch_shapes=[
            pltpu.VMEM((B, 1), jnp.float32),
            pltpu.VMEM((B, 1), jnp.int32),
            pltpu.VMEM((B, 1), jnp.float32),
            pltpu.VMEM((B, 1), jnp.int32),
        ],
    )(tops, topi, M, Z, ks, ps, logits, q)


def kernel(logits, top_ks, top_ps, q):
    tops_flat, topi_flat = _row_tops(logits.reshape(-1))
    tops = tops_flat.reshape(B, K64)
    topi = topi_flat.reshape(B, K64)
    M, Z = _row_stats(logits)
    masked, idx = _finalize(
        tops, topi, M, Z,
        top_ks.astype(jnp.int32).reshape(B, 1),
        top_ps.reshape(B, 1),
        logits, q)
    return idx.reshape(B), masked


# speedup vs baseline: 2.0680x; 2.0680x over previous
import functools

import jax
import jax.numpy as jnp
from jax import lax
from jax.experimental import pallas as pl
from jax.experimental.pallas import tpu as pltpu
from jax.experimental.pallas import tpu_sc as plsc

B = 128
V = 100000
K64 = 64
NEG = -1e30
EPS = 1e-8
NINF = float("-inf")
PINF = float("inf")


WA = 4096
NA = -(-V // WA)


def _stats_body(x_ref, m_ref, z_ref, ms, zs):
    pid = pl.program_id(0)

    @pl.when(pid == 0)
    def _():
        ms[...] = jnp.full_like(ms[...], NINF)
        zs[...] = jnp.zeros_like(zs[...])

    x = x_ref[...]
    cols = pid * WA + lax.broadcasted_iota(jnp.int32, (B, WA), 1)
    xv = jnp.where(cols < V, x, NINF)
    bm = jnp.max(xv, axis=1, keepdims=True)
    m_old = ms[...]
    m_new = jnp.maximum(m_old, bm)
    bz = jnp.sum(jnp.exp(xv - m_new), axis=1, keepdims=True)
    zs[...] = zs[...] * jnp.exp(m_old - m_new) + bz
    ms[...] = m_new
    m_ref[...] = ms[...]
    z_ref[...] = zs[...]


def _row_stats(logits):
    return pl.pallas_call(
        _stats_body,
        grid=(NA,),
        in_specs=[pl.BlockSpec((B, WA), lambda i: (0, i))],
        out_specs=[
            pl.BlockSpec((B, 1), lambda i: (0, 0)),
            pl.BlockSpec((B, 1), lambda i: (0, 0)),
        ],
        out_shape=[
            jax.ShapeDtypeStruct((B, 1), jnp.float32),
            jax.ShapeDtypeStruct((B, 1), jnp.float32),
        ],
        scratch_shapes=[
            pltpu.VMEM((B, 1), jnp.float32),
            pltpu.VMEM((B, 1), jnp.float32),
        ],
    )(logits)


NC = 2
NS = 16
NW = NC * NS
ROWS_PER = B // NW
GRP = 5
NGRP = V // (16 * GRP)
APG = 3
APV = APG * GRP
AP0 = K64
CANDV = (AP0 // 16) + APV
CAND = CANDV * 16


def _scalar(vec):
    return lax.squeeze(lax.slice(vec, (0,), (1,)), dimensions=(0,))


def _splat_f(s):
    return jnp.full((16,), s, jnp.float32)


def _splat_i(s):
    return jnp.full((16,), s, jnp.int32)


def _topk_body(logits_hbm, tops_hbm, topi_hbm, buf, cand, candi,
               topsv, topsi):
    wid = lax.axis_index("s") * NC + lax.axis_index("c")
    lane = lax.iota(jnp.int32, 16)

    def _consolidate(_):
        def find_body(j, carry):
            t_cur, t64 = carry
            tc = _splat_f(t_cur)

            def mx_body(k, mc):
                m, c = mc
                v = cand[pl.ds(k * 16, 16)]
                ge = v >= tc
                m = jnp.maximum(m, jnp.where(ge, _splat_f(NINF), v))
                c = c + jnp.where(ge, 1, 0).astype(jnp.int32)
                return m, c

            mvec, cvec = lax.fori_loop(0, CANDV, mx_body,
                                       (_splat_f(NINF), _splat_i(0)))
            nm = jnp.max(mvec)
            rank_before = jnp.sum(cvec)
            t64 = jnp.where(rank_before < K64, nm, t64)
            return nm, t64

        _, t64 = lax.fori_loop(
            0, K64, find_body, (jnp.float32(PINF), jnp.float32(NINF)))
        t64s = _splat_f(t64)

        def g_body(k, ge):
            g, e = ge
            v = cand[pl.ds(k * 16, 16)]
            g = g + jnp.where(v > t64s, 1, 0).astype(jnp.int32)
            e = e + jnp.where(v == t64s, 1, 0).astype(jnp.int32)
            return g, e

        g, e = lax.fori_loop(0, CANDV, g_body, (_splat_i(0), _splat_i(0)))
        skip = jnp.sum(e) - (K64 - jnp.sum(g))

        def comp_body(k, carry):
            w, eq_seen = carry
            v = cand[pl.ds(k * 16, 16)]
            iv = candi[pl.ds(k * 16, 16)]
            gm = v > t64s
            em = v == t64s
            pos = plsc.cumsum(jnp.where(em, 1, 0).astype(jnp.int32))
            pos = pos + _splat_i(eq_seen)
            take = jnp.logical_or(
                gm, jnp.logical_and(em, pos > _splat_i(skip)))
            plsc.store_compressed(cand.at[pl.ds(w, 16)], v, mask=take)
            plsc.store_compressed(candi.at[pl.ds(w, 16)], iv, mask=take)
            nw = w + jnp.sum(jnp.where(take, 1, 0).astype(jnp.int32))
            return nw, eq_seen + jnp.sum(jnp.where(em, 1, 0).astype(jnp.int32))

        lax.fori_loop(0, CANDV, comp_body, (jnp.int32(0), jnp.int32(0)))

        def clr_body(k, c):
            cand[pl.ds(AP0 + k * 16, 16)] = _splat_f(NINF)
            return c

        lax.fori_loop(0, APV, clr_body, jnp.int32(0))
        return t64

    def row_body(r, c0):
        row = wid * ROWS_PER + r

        def initc(k, c):
            cand[pl.ds(k * 16, 16)] = _splat_f(NINF)
            return c

        lax.fori_loop(0, CANDV, initc, jnp.int32(0))
        pltpu.sync_copy(logits_hbm.at[pl.ds(row * V, V)], buf)

        def scan_body(i, carry):
            tv, cnt = carry
            vs = [buf[pl.ds((i * GRP + u) * 16, 16)] for u in range(GRP)]
            m = vs[0]
            for u in range(1, GRP):
                m = jnp.maximum(m, vs[u])
            hit = jnp.any(m >= tv)

            def do_append(tv, cnt):
                for u in range(GRP):
                    cand[pl.ds(AP0 + (cnt + u) * 16, 16)] = vs[u]
                    candi[pl.ds(AP0 + (cnt + u) * 16, 16)] = (
                        _splat_i((i * GRP + u) * 16) + lane)
                cnt = cnt + GRP

                def run_cons(_):
                    return _splat_f(_consolidate(0)), jnp.int32(0)

                return lax.cond(cnt >= APV, run_cons,
                                lambda _: (tv, cnt), 0)

            return lax.cond(hit, do_append, lambda tv, cnt: (tv, cnt),
                            tv, cnt)

        lax.fori_loop(0, NGRP, scan_body, (_splat_f(NINF), jnp.int32(0)))
        _consolidate(0)

        def ex_body(j, carry):
            cur, rem, pidx = carry

            def adv(_):
                cs = _splat_f(cur)
                m = _splat_f(NINF)
                for k in range(K64 // 16):
                    v = cand[pl.ds(k * 16, 16)]
                    m = jnp.maximum(m, jnp.where(v < cs, v, _splat_f(NINF)))
                nm = jnp.max(m)
                nms = _splat_f(nm)
                cvec = _splat_i(0)
                for k in range(K64 // 16):
                    v = cand[pl.ds(k * 16, 16)]
                    cvec = cvec + jnp.where(v == nms, 1, 0).astype(jnp.int32)
                return nm, jnp.sum(cvec), jnp.int32(2 ** 30)

            cur, rem, pidx = lax.cond(rem == 0, adv,
                                      lambda _: (cur, rem, pidx), 0)
            curs = _splat_f(cur)
            pv = _splat_i(pidx)
            mi = _splat_i(-1)
            for k in range(K64 // 16):
                v = cand[pl.ds(k * 16, 16)]
                iv = candi[pl.ds(k * 16, 16)]
                sel = jnp.logical_and(v == curs, iv < pv)
                mi = jnp.maximum(mi, jnp.where(sel, iv, _splat_i(-1)))
            eidx = jnp.max(mi)
            plsc.store_scatter(topsv, [_splat_i(j)], curs, mask=lane == 0)
            plsc.store_scatter(topsi, [_splat_i(j)], _splat_i(eidx),
                               mask=lane == 0)
            return cur, rem - 1, eidx

        lax.fori_loop(0, K64, ex_body,
                      (jnp.float32(PINF), jnp.int32(0), jnp.int32(2 ** 30)))
        pltpu.sync_copy(topsv, tops_hbm.at[pl.ds(row * K64, K64)])
        pltpu.sync_copy(topsi, topi_hbm.at[pl.ds(row * K64, K64)])
        return c0

    lax.fori_loop(0, ROWS_PER, row_body, jnp.int32(0))


def _row_tops(logits_flat):
    mesh = plsc.VectorSubcoreMesh(
        core_axis_name="c", subcore_axis_name="s",
        num_cores=NC, num_subcores=NS)
    return pl.kernel(
        _topk_body,
        out_type=(jax.ShapeDtypeStruct((B * K64,), jnp.float32),
                  jax.ShapeDtypeStruct((B * K64,), jnp.int32)),
        mesh=mesh,
        compiler_params=pltpu.CompilerParams(needs_layout_passes=False),
        scratch_types=[
            pltpu.VMEM((V,), jnp.float32),
            pltpu.VMEM((CAND,), jnp.float32),
            pltpu.VMEM((CAND,), jnp.int32),
            pltpu.VMEM((K64,), jnp.float32),
            pltpu.VMEM((K64,), jnp.int32),
        ],
    )(logits_flat)


WD = 4096
ND = -(-V // WD)


def _final_body(tops_ref, topi_ref, m_ref, z_ref, ks_ref, ps_ref, x_ref,
                q_ref, out_ref, idx_ref, ts, cs_s, bv, bi):
    pid = pl.program_id(0)

    @pl.when(pid == 0)
    def _():
        tops = tops_ref[...]
        topi = topi_ref[...]
        M = m_ref[...]
        Z = z_ref[...]
        p = jnp.exp(tops - M) / Z
        r = lax.broadcasted_iota(jnp.int32, (K64, K64), 0)
        c = lax.broadcasted_iota(jnp.int32, (K64, K64), 1)
        ut = (r < c).astype(jnp.float32)
        excl = jnp.dot(p, ut, preferred_element_type=jnp.float32)
        ranks = lax.broadcasted_iota(jnp.int32, (B, K64), 1)
        keep = (ranks < ks_ref[...]) & (excl < ps_ref[...])
        m = jnp.sum(keep.astype(jnp.int32), axis=1, keepdims=True)
        last = ranks == (m - 1)
        ts[...] = jnp.max(jnp.where(last, tops, NINF), axis=1, keepdims=True)
        cs_s[...] = jnp.max(jnp.where(last, topi, -1), axis=1, keepdims=True)
        bv[...] = jnp.full((B, 1), -1.0, jnp.float32)
        bi[...] = jnp.zeros((B, 1), jnp.int32)

    x = x_ref[...]
    q = q_ref[...]
    cols = pid * WD + lax.broadcasted_iota(jnp.int32, (B, WD), 1)
    tsv = ts[...]
    kept = ((x > tsv) | ((x == tsv) & (cols >= cs_s[...]))) & (cols < V)
    out_ref[...] = jnp.where(kept, x, NEG)
    val = jnp.where(kept, jnp.exp(x - m_ref[...]) / jnp.maximum(q, EPS),
                    jnp.float32(-1.0))
    cmax = jnp.max(val, axis=1, keepdims=True)
    cidx = jnp.min(jnp.where(val == cmax, cols, jnp.int32(V)),
                   axis=1, keepdims=True)
    better = cmax > bv[...]
    bv[...] = jnp.where(better, cmax, bv[...])
    bi[...] = jnp.where(better, cidx, bi[...])
    idx_ref[...] = bi[...]


def _finalize(tops, topi, M, Z, ks, ps, logits, q):
    return pl.pallas_call(
        _final_body,
        grid=(ND,),
        in_specs=[
            pl.BlockSpec((B, K64), lambda i: (0, 0)),
            pl.BlockSpec((B, K64), lambda i: (0, 0)),
            pl.BlockSpec((B, 1), lambda i: (0, 0)),
            pl.BlockSpec((B, 1), lambda i: (0, 0)),
            pl.BlockSpec((B, 1), lambda i: (0, 0)),
            pl.BlockSpec((B, 1), lambda i: (0, 0)),
            pl.BlockSpec((B, WD), lambda i: (0, i)),
            pl.BlockSpec((B, WD), lambda i: (0, i)),
        ],
        out_specs=[
            pl.BlockSpec((B, WD), lambda i: (0, i)),
            pl.BlockSpec((B, 1), lambda i: (0, 0)),
        ],
        out_shape=[
            jax.ShapeDtypeStruct((B, V), jnp.float32),
            jax.ShapeDtypeStruct((B, 1), jnp.int32),
        ],
        scratch_shapes=[
            pltpu.VMEM((B, 1), jnp.float32),
            pltpu.VMEM((B, 1), jnp.int32),
            pltpu.VMEM((B, 1), jnp.float32),
            pltpu.VMEM((B, 1), jnp.int32),
        ],
    )(tops, topi, M, Z, ks, ps, logits, q)


def kernel(logits, top_ks, top_ps, q):
    tops_flat, topi_flat = _row_tops(logits.reshape(-1))
    tops = tops_flat.reshape(B, K64)
    topi = topi_flat.reshape(B, K64)
    M, Z = _row_stats(logits)
    masked, idx = _finalize(
        tops, topi, M, Z,
        top_ks.astype(jnp.int32).reshape(B, 1),
        top_ps.reshape(B, 1),
        logits, q)
    return idx.reshape(B), masked


# speedup vs baseline: 2.1282x; 1.0291x over previous
import functools

import jax
import jax.numpy as jnp
from jax import lax
from jax.experimental import pallas as pl
from jax.experimental.pallas import tpu as pltpu
from jax.experimental.pallas import tpu_sc as plsc

B = 128
V = 100000
K64 = 64
NEG = -1e30
EPS = 1e-8
NINF = float("-inf")
PINF = float("inf")


WA = 4096
NA = -(-V // WA)


def _stats_body(x_ref, m_ref, z_ref, ms, zs):
    pid = pl.program_id(0)

    @pl.when(pid == 0)
    def _():
        ms[...] = jnp.full_like(ms[...], NINF)
        zs[...] = jnp.zeros_like(zs[...])

    x = x_ref[...]
    cols = pid * WA + lax.broadcasted_iota(jnp.int32, (B, WA), 1)
    xv = jnp.where(cols < V, x, NINF)
    bm = jnp.max(xv, axis=1, keepdims=True)
    m_old = ms[...]
    m_new = jnp.maximum(m_old, bm)
    bz = jnp.sum(jnp.exp(xv - m_new), axis=1, keepdims=True)
    zs[...] = zs[...] * jnp.exp(m_old - m_new) + bz
    ms[...] = m_new
    m_ref[...] = ms[...]
    z_ref[...] = zs[...]


def _row_stats(logits):
    return pl.pallas_call(
        _stats_body,
        grid=(NA,),
        in_specs=[pl.BlockSpec((B, WA), lambda i: (0, i))],
        out_specs=[
            pl.BlockSpec((B, 1), lambda i: (0, 0)),
            pl.BlockSpec((B, 1), lambda i: (0, 0)),
        ],
        out_shape=[
            jax.ShapeDtypeStruct((B, 1), jnp.float32),
            jax.ShapeDtypeStruct((B, 1), jnp.float32),
        ],
        scratch_shapes=[
            pltpu.VMEM((B, 1), jnp.float32),
            pltpu.VMEM((B, 1), jnp.float32),
        ],
    )(logits)


NC = 2
NS = 16
NW = NC * NS
ROWS_PER = B // NW
GRP = 5
SUP = 5
NGRP = V // (16 * GRP)
APG = 3
APV = APG * GRP
AP0 = K64
CANDV = (AP0 // 16) + APV
CAND = CANDV * 16


def _scalar(vec):
    return lax.squeeze(lax.slice(vec, (0,), (1,)), dimensions=(0,))


def _splat_f(s):
    return jnp.full((16,), s, jnp.float32)


def _splat_i(s):
    return jnp.full((16,), s, jnp.int32)


def _topk_body(logits_hbm, tops_hbm, topi_hbm, buf, cand, candi,
               topsv, topsi):
    wid = lax.axis_index("s") * NC + lax.axis_index("c")
    lane = lax.iota(jnp.int32, 16)

    def _consolidate(_):
        def find_body(j, carry):
            t_cur, t64 = carry
            tc = _splat_f(t_cur)

            def mx_body(k, mc):
                m, c = mc
                v = cand[pl.ds(k * 16, 16)]
                ge = v >= tc
                m = jnp.maximum(m, jnp.where(ge, _splat_f(NINF), v))
                c = c + jnp.where(ge, 1, 0).astype(jnp.int32)
                return m, c

            mvec, cvec = lax.fori_loop(0, CANDV, mx_body,
                                       (_splat_f(NINF), _splat_i(0)))
            nm = jnp.max(mvec)
            rank_before = jnp.sum(cvec)
            t64 = jnp.where(rank_before < K64, nm, t64)
            return nm, t64

        _, t64 = lax.fori_loop(
            0, K64, find_body, (jnp.float32(PINF), jnp.float32(NINF)))
        t64s = _splat_f(t64)

        def g_body(k, ge):
            g, e = ge
            v = cand[pl.ds(k * 16, 16)]
            g = g + jnp.where(v > t64s, 1, 0).astype(jnp.int32)
            e = e + jnp.where(v == t64s, 1, 0).astype(jnp.int32)
            return g, e

        g, e = lax.fori_loop(0, CANDV, g_body, (_splat_i(0), _splat_i(0)))
        skip = jnp.sum(e) - (K64 - jnp.sum(g))

        def comp_body(k, carry):
            w, eq_seen = carry
            v = cand[pl.ds(k * 16, 16)]
            iv = candi[pl.ds(k * 16, 16)]
            gm = v > t64s
            em = v == t64s
            pos = plsc.cumsum(jnp.where(em, 1, 0).astype(jnp.int32))
            pos = pos + _splat_i(eq_seen)
            take = jnp.logical_or(
                gm, jnp.logical_and(em, pos > _splat_i(skip)))
            plsc.store_compressed(cand.at[pl.ds(w, 16)], v, mask=take)
            plsc.store_compressed(candi.at[pl.ds(w, 16)], iv, mask=take)
            nw = w + jnp.sum(jnp.where(take, 1, 0).astype(jnp.int32))
            return nw, eq_seen + jnp.sum(jnp.where(em, 1, 0).astype(jnp.int32))

        lax.fori_loop(0, CANDV, comp_body, (jnp.int32(0), jnp.int32(0)))

        def clr_body(k, c):
            cand[pl.ds(AP0 + k * 16, 16)] = _splat_f(NINF)
            return c

        lax.fori_loop(0, APV, clr_body, jnp.int32(0))
        return t64

    def row_body(r, c0):
        row = wid * ROWS_PER + r

        def initc(k, c):
            cand[pl.ds(k * 16, 16)] = _splat_f(NINF)
            return c

        lax.fori_loop(0, CANDV, initc, jnp.int32(0))
        pltpu.sync_copy(logits_hbm.at[pl.ds(row * V, V)], buf)

        def scan_body(j, carry):
            tv, cnt = carry
            vss = []
            msub = []
            mall = None
            for g in range(SUP):
                vs = [buf[pl.ds(((j * SUP + g) * GRP + u) * 16, 16)]
                      for u in range(GRP)]
                m = vs[0]
                for u in range(1, GRP):
                    m = jnp.maximum(m, vs[u])
                vss.append(vs)
                msub.append(m)
                mall = m if mall is None else jnp.maximum(mall, m)

            def sup_hit(tv, cnt):
                for g in range(SUP):
                    def do_append(tv, cnt, g=g):
                        for u in range(GRP):
                            cand[pl.ds(AP0 + (cnt + u) * 16, 16)] = vss[g][u]
                            candi[pl.ds(AP0 + (cnt + u) * 16, 16)] = (
                                _splat_i(((j * SUP + g) * GRP + u) * 16)
                                + lane)
                        cnt = cnt + GRP

                        def run_cons(_):
                            return _splat_f(_consolidate(0)), jnp.int32(0)

                        return lax.cond(cnt >= APV, run_cons,
                                        lambda _: (tv, cnt), 0)

                    tv, cnt = lax.cond(jnp.any(msub[g] >= tv), do_append,
                                       lambda tv, cnt: (tv, cnt), tv, cnt)
                return tv, cnt

            return lax.cond(jnp.any(mall >= tv), sup_hit,
                            lambda tv, cnt: (tv, cnt), tv, cnt)

        lax.fori_loop(0, NGRP // SUP, scan_body,
                      (_splat_f(NINF), jnp.int32(0)))
        _consolidate(0)

        def ex_body(j, carry):
            cur, rem, pidx = carry

            def adv(_):
                cs = _splat_f(cur)
                m = _splat_f(NINF)
                for k in range(K64 // 16):
                    v = cand[pl.ds(k * 16, 16)]
                    m = jnp.maximum(m, jnp.where(v < cs, v, _splat_f(NINF)))
                nm = jnp.max(m)
                nms = _splat_f(nm)
                cvec = _splat_i(0)
                for k in range(K64 // 16):
                    v = cand[pl.ds(k * 16, 16)]
                    cvec = cvec + jnp.where(v == nms, 1, 0).astype(jnp.int32)
                return nm, jnp.sum(cvec), jnp.int32(2 ** 30)

            cur, rem, pidx = lax.cond(rem == 0, adv,
                                      lambda _: (cur, rem, pidx), 0)
            curs = _splat_f(cur)
            pv = _splat_i(pidx)
            mi = _splat_i(-1)
            for k in range(K64 // 16):
                v = cand[pl.ds(k * 16, 16)]
                iv = candi[pl.ds(k * 16, 16)]
                sel = jnp.logical_and(v == curs, iv < pv)
                mi = jnp.maximum(mi, jnp.where(sel, iv, _splat_i(-1)))
            eidx = jnp.max(mi)
            plsc.store_scatter(topsv, [_splat_i(j)], curs, mask=lane == 0)
            plsc.store_scatter(topsi, [_splat_i(j)], _splat_i(eidx),
                               mask=lane == 0)
            return cur, rem - 1, eidx

        lax.fori_loop(0, K64, ex_body,
                      (jnp.float32(PINF), jnp.int32(0), jnp.int32(2 ** 30)))
        pltpu.sync_copy(topsv, tops_hbm.at[pl.ds(row * K64, K64)])
        pltpu.sync_copy(topsi, topi_hbm.at[pl.ds(row * K64, K64)])
        return c0

    lax.fori_loop(0, ROWS_PER, row_body, jnp.int32(0))


def _row_tops(logits_flat):
    mesh = plsc.VectorSubcoreMesh(
        core_axis_name="c", subcore_axis_name="s",
        num_cores=NC, num_subcores=NS)
    return pl.kernel(
        _topk_body,
        out_type=(jax.ShapeDtypeStruct((B * K64,), jnp.float32),
                  jax.ShapeDtypeStruct((B * K64,), jnp.int32)),
        mesh=mesh,
        compiler_params=pltpu.CompilerParams(needs_layout_passes=False),
        scratch_types=[
            pltpu.VMEM((V,), jnp.float32),
            pltpu.VMEM((CAND,), jnp.float32),
            pltpu.VMEM((CAND,), jnp.int32),
            pltpu.VMEM((K64,), jnp.float32),
            pltpu.VMEM((K64,), jnp.int32),
        ],
    )(logits_flat)


WD = 4096
ND = -(-V // WD)


def _final_body(tops_ref, topi_ref, m_ref, z_ref, ks_ref, ps_ref, x_ref,
                q_ref, out_ref, idx_ref, ts, cs_s, bv, bi):
    pid = pl.program_id(0)

    @pl.when(pid == 0)
    def _():
        tops = tops_ref[...]
        topi = topi_ref[...]
        M = m_ref[...]
        Z = z_ref[...]
        p = jnp.exp(tops - M) / Z
        r = lax.broadcasted_iota(jnp.int32, (K64, K64), 0)
        c = lax.broadcasted_iota(jnp.int32, (K64, K64), 1)
        ut = (r < c).astype(jnp.float32)
        excl = jnp.dot(p, ut, preferred_element_type=jnp.float32)
        ranks = lax.broadcasted_iota(jnp.int32, (B, K64), 1)
        keep = (ranks < ks_ref[...]) & (excl < ps_ref[...])
        m = jnp.sum(keep.astype(jnp.int32), axis=1, keepdims=True)
        last = ranks == (m - 1)
        ts[...] = jnp.max(jnp.where(last, tops, NINF), axis=1, keepdims=True)
        cs_s[...] = jnp.max(jnp.where(last, topi, -1), axis=1, keepdims=True)
        bv[...] = jnp.full((B, 1), -1.0, jnp.float32)
        bi[...] = jnp.zeros((B, 1), jnp.int32)

    x = x_ref[...]
    q = q_ref[...]
    cols = pid * WD + lax.broadcasted_iota(jnp.int32, (B, WD), 1)
    tsv = ts[...]
    kept = ((x > tsv) | ((x == tsv) & (cols >= cs_s[...]))) & (cols < V)
    out_ref[...] = jnp.where(kept, x, NEG)
    val = jnp.where(kept, jnp.exp(x - m_ref[...]) / jnp.maximum(q, EPS),
                    jnp.float32(-1.0))
    cmax = jnp.max(val, axis=1, keepdims=True)
    cidx = jnp.min(jnp.where(val == cmax, cols, jnp.int32(V)),
                   axis=1, keepdims=True)
    better = cmax > bv[...]
    bv[...] = jnp.where(better, cmax, bv[...])
    bi[...] = jnp.where(better, cidx, bi[...])
    idx_ref[...] = bi[...]


def _finalize(tops, topi, M, Z, ks, ps, logits, q):
    return pl.pallas_call(
        _final_body,
        grid=(ND,),
        in_specs=[
            pl.BlockSpec((B, K64), lambda i: (0, 0)),
            pl.BlockSpec((B, K64), lambda i: (0, 0)),
            pl.BlockSpec((B, 1), lambda i: (0, 0)),
            pl.BlockSpec((B, 1), lambda i: (0, 0)),
            pl.BlockSpec((B, 1), lambda i: (0, 0)),
            pl.BlockSpec((B, 1), lambda i: (0, 0)),
            pl.BlockSpec((B, WD), lambda i: (0, i)),
            pl.BlockSpec((B, WD), lambda i: (0, i)),
        ],
        out_specs=[
            pl.BlockSpec((B, WD), lambda i: (0, i)),
            pl.BlockSpec((B, 1), lambda i: (0, 0)),
        ],
        out_shape=[
            jax.ShapeDtypeStruct((B, V), jnp.float32),
            jax.ShapeDtypeStruct((B, 1), jnp.int32),
        ],
        scratch_shapes=[
            pltpu.VMEM((B, 1), jnp.float32),
            pltpu.VMEM((B, 1), jnp.int32),
            pltpu.VMEM((B, 1), jnp.float32),
            pltpu.VMEM((B, 1), jnp.int32),
        ],
    )(tops, topi, M, Z, ks, ps, logits, q)


def kernel(logits, top_ks, top_ps, q):
    tops_flat, topi_flat = _row_tops(logits.reshape(-1))
    tops = tops_flat.reshape(B, K64)
    topi = topi_flat.reshape(B, K64)
    M, Z = _row_stats(logits)
    masked, idx = _finalize(
        tops, topi, M, Z,
        top_ks.astype(jnp.int32).reshape(B, 1),
        top_ps.reshape(B, 1),
        logits, q)
    return idx.reshape(B), masked
